# baseline (speedup 1.0000x reference)
import functools

import jax
import jax.numpy as jnp
from jax import lax
from jax.experimental import pallas as pl
from jax.experimental.pallas import tpu as pltpu
from jax.experimental.pallas import tpu_sc as plsc

_NC = 2
_NS = 16
_NW = _NC * _NS
_L = 16
_BLK = 128


def _mesh():
    return plsc.VectorSubcoreMesh(
        core_axis_name="c", subcore_axis_name="s",
        num_cores=_NC, num_subcores=_NS)


def _wid():
    return lax.axis_index("s") * _NC + lax.axis_index("c")


def _transpose_block(src, dst, nrows, d):
    lanes = lax.iota(jnp.int32, _L)
    perms = [(lanes + k) & (_L - 1) for k in range(_L)]

    @pl.loop(0, nrows // _L, unroll=2)
    def _i(it):
        rowv = it * _L + lanes

        @pl.loop(0, d // _L, unroll=4)
        def _j(jt):
            d0 = jt * _L
            for k in range(_L):
                colv = d0 + perms[k]
                vals = plsc.load_gather(src, [colv, rowv])
                plsc.store_scatter(dst, [rowv, colv], vals)


def _relayout_table(wt, tail128, v, d):
    nfull = v // _BLK
    tail = v - nfull * _BLK
    per_w = nfull // _NW
    extra = nfull - per_w * _NW

    @functools.partial(
        pl.kernel,
        out_type=jax.ShapeDtypeStruct((v, _BLK), jnp.float32),
        mesh=_mesh(),
        scratch_types=[
            pltpu.VMEM((d, _BLK), jnp.float32),
            pltpu.VMEM((d, _BLK), jnp.float32),
            pltpu.VMEM((_BLK, _BLK), jnp.float32),
            pltpu.VMEM((_BLK, _BLK), jnp.float32),
            pltpu.SemaphoreType.DMA,
            pltpu.SemaphoreType.DMA,
            pltpu.SemaphoreType.DMA,
            pltpu.SemaphoreType.DMA,
        ],
        compiler_params=pltpu.CompilerParams(needs_layout_passes=False),
    )
    def body(wt_hbm, tail_hbm, tab_hbm, stage0, stage1, tblk0, tblk1,
             rsem0, rsem1, wsem0, wsem1):
        stage = (stage0, stage1)
        tblk = (tblk0, tblk1)
        rsem = (rsem0, rsem1)
        wsem = (wsem0, wsem1)
        wid = _wid()
        off = wid * per_w + jnp.minimum(wid, extra)
        nb = per_w + 2

        def c0_of(j):
            return jnp.minimum((off + j) * _BLK, (nfull - 1) * _BLK)

        def read(j, b):
            pltpu.async_copy(
                wt_hbm.at[:, pl.ds(c0_of(j), _BLK)], stage[b], rsem[b])

        def wait_read(b):
            pltpu.make_async_copy(
                wt_hbm.at[:, pl.ds(0, _BLK)], stage[b], rsem[b]).wait()

        def write(j, b):
            pltpu.async_copy(
                tblk[b], tab_hbm.at[pl.ds(c0_of(j), _BLK)], wsem[b])

        def wait_write(b):
            pltpu.make_async_copy(
                tblk[b], tab_hbm.at[pl.ds(0, _BLK)], wsem[b]).wait()

        def step(j, b):
            wait_read(b)

            @pl.when(j > 1)
            def _():
                wait_write(b)
            _transpose_block(stage[b], tblk[b], _BLK, d)
            write(j, b)

            @pl.when(j + 2 < nb)
            def _():
                read(j + 2, b)

        read(0, 0)
        read(1, 1)

        @pl.loop(0, nb, step=2)
        def _per_blk(j):
            step(j, 0)
            step(j + 1, 1)

        wait_write(0)
        wait_write(1)

        if tail:
            @pl.when(wid == _NW - 1)
            def _tail():
                pltpu.sync_copy(tail_hbm.at[pl.ds(0, tail)],
                                tblk0.at[pl.ds(0, tail)])
                pltpu.sync_copy(tblk0.at[pl.ds(0, tail)],
                                tab_hbm.at[pl.ds(nfull * _BLK, tail)])

    return body(wt, tail128)


def _embed_gather(xt, tab, bsz, h, d):
    bw = bsz // _NW

    @functools.partial(
        pl.kernel,
        out_type=jax.ShapeDtypeStruct((h, d, bsz), jnp.float32),
        mesh=_mesh(),
        scratch_types=[
            pltpu.VMEM((h, bw), jnp.int32),
            pltpu.VMEM((bw, _BLK), jnp.float32),
            pltpu.VMEM((bw, _BLK), jnp.float32),
            pltpu.VMEM((d, bw), jnp.float32),
            pltpu.VMEM((d, bw), jnp.float32),
            pltpu.SemaphoreType.DMA,
            pltpu.SemaphoreType.DMA,
            pltpu.SemaphoreType.DMA,
            pltpu.SemaphoreType.DMA,
        ],
        compiler_params=pltpu.CompilerParams(needs_layout_passes=False),
    )
    def body(xt_hbm, tab_hbm, out_hbm, idxv, pair0, pair1,
             outb0, outb1, gsem0, gsem1, osem0, osem1):
        pair = (pair0, pair1)
        outbuf = (outb0, outb1)
        gsem = (gsem0, gsem1)
        osem = (osem0, osem1)
        wid = _wid()
        b0 = wid * bw

        pltpu.sync_copy(xt_hbm.at[:, pl.ds(b0, bw)], idxv)

        lanes = lax.iota(jnp.int32, _L)
        rowvecs = [lanes + k * _L for k in range(bw // _L)]

        def fire(l, b):
            pltpu.async_copy(tab_hbm.at[idxv.at[l]], pair[b], gsem[b])

        def drain(b):
            pltpu.make_async_copy(
                tab_hbm.at[pl.ds(0, bw)], pair[b], gsem[b]).wait()

        def emit(l, b):
            drain(b)

            @pl.when(l > 1)
            def _():
                pltpu.make_async_copy(
                    outbuf[b], out_hbm.at[0, :, pl.ds(0, bw)],
                    osem[b]).wait()
            _transpose_block(pair[b], outbuf[b], d, bw)
            pltpu.async_copy(
                outbuf[b], out_hbm.at[l, :, pl.ds(b0, bw)], osem[b])

        fire(0, 0)

        @pl.loop(0, h, step=2)
        def _pair_l(l):
            fire(l + 1, 1)
            emit(l, 0)

            @pl.when(l + 2 < h)
            def _():
                fire(l + 2, 0)
            emit(l + 1, 1)

        for b in (0, 1):
            pltpu.make_async_copy(
                outbuf[b], out_hbm.at[0, :, pl.ds(0, bw)], osem[b]).wait()

    return body(xt, tab)


def kernel(x, embed_weights):
    bsz, h = x.shape
    v, d = embed_weights.shape
    xt = x.T.astype(jnp.int32)
    wt = embed_weights.T
    nfull = v // _BLK
    tail128 = jnp.pad(embed_weights[nfull * _BLK:],
                      ((0, _BLK - (v - nfull * _BLK)), (0, _BLK - d)))
    tab = _relayout_table(wt, tail128, v, d)
    o = _embed_gather(xt, tab, bsz, h, d)
    return jnp.transpose(o, (2, 0, 1))


# speedup vs baseline: 1.1494x; 1.1494x over previous
import functools

import jax
import jax.numpy as jnp
from jax import lax
from jax.experimental import pallas as pl
from jax.experimental.pallas import tpu as pltpu
from jax.experimental.pallas import tpu_sc as plsc

_NC = 2
_NS = 16
_NW = _NC * _NS
_L = 16
_BLK = 128


def _mesh():
    return plsc.VectorSubcoreMesh(
        core_axis_name="c", subcore_axis_name="s",
        num_cores=_NC, num_subcores=_NS)


def _wid():
    return lax.axis_index("s") * _NC + lax.axis_index("c")


def _transpose_block(src, dst, nrows, d):
    lanes = lax.iota(jnp.int32, _L)
    perms = [(lanes + k) & (_L - 1) for k in range(_L)]

    @pl.loop(0, nrows // _L)
    def _i(it):
        rowv = it * _L + lanes

        @pl.loop(0, d // _L)
        def _j(jt):
            d0 = jt * _L
            for k in range(_L):
                colv = d0 + perms[k]
                vals = plsc.load_gather(src, [colv, rowv])
                plsc.store_scatter(dst, [rowv, colv], vals)


def _relayout_table(wt, tail128, v, d):
    nfull = v // _BLK
    tail = v - nfull * _BLK
    per_w = nfull // _NW
    extra = nfull - per_w * _NW

    @functools.partial(
        pl.kernel,
        out_type=jax.ShapeDtypeStruct((v, _BLK), jnp.float32),
        mesh=_mesh(),
        scratch_types=[
            pltpu.VMEM((d, _BLK), jnp.float32),
            pltpu.VMEM((d, _BLK), jnp.float32),
            pltpu.VMEM((_BLK, _BLK), jnp.float32),
            pltpu.VMEM((_BLK, _BLK), jnp.float32),
            pltpu.SemaphoreType.DMA,
            pltpu.SemaphoreType.DMA,
            pltpu.SemaphoreType.DMA,
            pltpu.SemaphoreType.DMA,
        ],
        compiler_params=pltpu.CompilerParams(needs_layout_passes=False),
    )
    def body(wt_hbm, tail_hbm, tab_hbm, stage0, stage1, tblk0, tblk1,
             rsem0, rsem1, wsem0, wsem1):
        stage = (stage0, stage1)
        tblk = (tblk0, tblk1)
        rsem = (rsem0, rsem1)
        wsem = (wsem0, wsem1)
        wid = _wid()
        off = wid * per_w + jnp.minimum(wid, extra)
        nb = per_w + 2

        def c0_of(j):
            return jnp.minimum((off + j) * _BLK, (nfull - 1) * _BLK)

        def read(j, b):
            pltpu.async_copy(
                wt_hbm.at[:, pl.ds(c0_of(j), _BLK)], stage[b], rsem[b])

        def wait_read(b):
            pltpu.make_async_copy(
                wt_hbm.at[:, pl.ds(0, _BLK)], stage[b], rsem[b]).wait()

        def write(j, b):
            pltpu.async_copy(
                tblk[b], tab_hbm.at[pl.ds(c0_of(j), _BLK)], wsem[b])

        def wait_write(b):
            pltpu.make_async_copy(
                tblk[b], tab_hbm.at[pl.ds(0, _BLK)], wsem[b]).wait()

        def step(j, b):
            wait_read(b)

            @pl.when(j > 1)
            def _():
                wait_write(b)
            _transpose_block(stage[b], tblk[b], _BLK, d)
            write(j, b)

            @pl.when(j + 2 < nb)
            def _():
                read(j + 2, b)

        read(0, 0)
        read(1, 1)

        @pl.loop(0, nb, step=2)
        def _per_blk(j):
            step(j, 0)
            step(j + 1, 1)

        wait_write(0)
        wait_write(1)

        if tail:
            @pl.when(wid == _NW - 1)
            def _tail():
                pltpu.sync_copy(tail_hbm.at[pl.ds(0, tail)],
                                tblk0.at[pl.ds(0, tail)])
                pltpu.sync_copy(tblk0.at[pl.ds(0, tail)],
                                tab_hbm.at[pl.ds(nfull * _BLK, tail)])

    return body(wt, tail128)


def _embed_gather(xt, tab, bsz, h, d):
    bw = bsz // _NW

    @functools.partial(
        pl.kernel,
        out_type=jax.ShapeDtypeStruct((h, d, bsz), jnp.float32),
        mesh=_mesh(),
        scratch_types=[
            pltpu.VMEM((h, bw), jnp.int32),
            pltpu.VMEM((bw, _BLK), jnp.float32),
            pltpu.VMEM((bw, _BLK), jnp.float32),
            pltpu.VMEM((d, bw), jnp.float32),
            pltpu.VMEM((d, bw), jnp.float32),
            pltpu.SemaphoreType.DMA,
            pltpu.SemaphoreType.DMA,
            pltpu.SemaphoreType.DMA,
            pltpu.SemaphoreType.DMA,
        ],
        compiler_params=pltpu.CompilerParams(needs_layout_passes=False),
    )
    def body(xt_hbm, tab_hbm, out_hbm, idxv, pair0, pair1,
             outb0, outb1, gsem0, gsem1, osem0, osem1):
        pair = (pair0, pair1)
        outbuf = (outb0, outb1)
        gsem = (gsem0, gsem1)
        osem = (osem0, osem1)
        wid = _wid()
        b0 = wid * bw

        pltpu.sync_copy(xt_hbm.at[:, pl.ds(b0, bw)], idxv)

        lanes = lax.iota(jnp.int32, _L)
        rowvecs = [lanes + k * _L for k in range(bw // _L)]

        def fire(l, b):
            pltpu.async_copy(tab_hbm.at[idxv.at[l]], pair[b], gsem[b])

        def drain(b):
            pltpu.make_async_copy(
                tab_hbm.at[pl.ds(0, bw)], pair[b], gsem[b]).wait()

        def emit(l, b):
            drain(b)

            @pl.when(l > 1)
            def _():
                pltpu.make_async_copy(
                    outbuf[b], out_hbm.at[0, :, pl.ds(0, bw)],
                    osem[b]).wait()
            _transpose_block(pair[b], outbuf[b], d, bw)
            pltpu.async_copy(
                outbuf[b], out_hbm.at[l, :, pl.ds(b0, bw)], osem[b])

        fire(0, 0)

        @pl.loop(0, h, step=2)
        def _pair_l(l):
            fire(l + 1, 1)
            emit(l, 0)

            @pl.when(l + 2 < h)
            def _():
                fire(l + 2, 0)
            emit(l + 1, 1)

        for b in (0, 1):
            pltpu.make_async_copy(
                outbuf[b], out_hbm.at[0, :, pl.ds(0, bw)], osem[b]).wait()

    return body(xt, tab)


def kernel(x, embed_weights):
    bsz, h = x.shape
    v, d = embed_weights.shape
    xt = x.T.astype(jnp.int32)
    wt = embed_weights.T
    nfull = v // _BLK
    tail128 = jnp.pad(embed_weights[nfull * _BLK:],
                      ((0, _BLK - (v - nfull * _BLK)), (0, _BLK - d)))
    tab = _relayout_table(wt, tail128, v, d)
    o = _embed_gather(xt, tab, bsz, h, d)
    return jnp.transpose(o, (2, 0, 1))


# speedup vs baseline: 1.1735x; 1.0209x over previous
import functools

import jax
import jax.numpy as jnp
from jax import lax
from jax.experimental import pallas as pl
from jax.experimental.pallas import tpu as pltpu
from jax.experimental.pallas import tpu_sc as plsc

_NC = 2
_NS = 16
_NW = _NC * _NS
_L = 16
_BLK = 128


def _mesh():
    return plsc.VectorSubcoreMesh(
        core_axis_name="c", subcore_axis_name="s",
        num_cores=_NC, num_subcores=_NS)


def _wid():
    return lax.axis_index("s") * _NC + lax.axis_index("c")


def _transpose_block(src, dst, nrows, d):
    lanes = lax.iota(jnp.int32, _L)
    perms = [(lanes + k) & (_L - 1) for k in range(_L)]

    @pl.loop(0, nrows // _L)
    def _i(it):
        rowv = it * _L + lanes

        @pl.loop(0, d // _L, unroll=2)
        def _j(jt):
            d0 = jt * _L
            for k in range(_L):
                colv = d0 + perms[k]
                vals = plsc.load_gather(src, [colv, rowv])
                plsc.store_scatter(dst, [rowv, colv], vals)


def _relayout_table(wt, tail128, v, d):
    nfull = v // _BLK
    tail = v - nfull * _BLK
    per_w = nfull // _NW
    extra = nfull - per_w * _NW

    @functools.partial(
        pl.kernel,
        out_type=jax.ShapeDtypeStruct((v, _BLK), jnp.float32),
        mesh=_mesh(),
        scratch_types=[
            pltpu.VMEM((d, _BLK), jnp.float32),
            pltpu.VMEM((d, _BLK), jnp.float32),
            pltpu.VMEM((_BLK, _BLK), jnp.float32),
            pltpu.VMEM((_BLK, _BLK), jnp.float32),
            pltpu.SemaphoreType.DMA,
            pltpu.SemaphoreType.DMA,
            pltpu.SemaphoreType.DMA,
            pltpu.SemaphoreType.DMA,
        ],
        compiler_params=pltpu.CompilerParams(needs_layout_passes=False),
    )
    def body(wt_hbm, tail_hbm, tab_hbm, stage0, stage1, tblk0, tblk1,
             rsem0, rsem1, wsem0, wsem1):
        stage = (stage0, stage1)
        tblk = (tblk0, tblk1)
        rsem = (rsem0, rsem1)
        wsem = (wsem0, wsem1)
        wid = _wid()
        off = wid * per_w + jnp.minimum(wid, extra)
        nb = per_w + 2

        def c0_of(j):
            return jnp.minimum((off + j) * _BLK, (nfull - 1) * _BLK)

        def read(j, b):
            pltpu.async_copy(
                wt_hbm.at[:, pl.ds(c0_of(j), _BLK)], stage[b], rsem[b])

        def wait_read(b):
            pltpu.make_async_copy(
                wt_hbm.at[:, pl.ds(0, _BLK)], stage[b], rsem[b]).wait()

        def write(j, b):
            pltpu.async_copy(
                tblk[b], tab_hbm.at[pl.ds(c0_of(j), _BLK)], wsem[b])

        def wait_write(b):
            pltpu.make_async_copy(
                tblk[b], tab_hbm.at[pl.ds(0, _BLK)], wsem[b]).wait()

        def step(j, b):
            wait_read(b)

            @pl.when(j > 1)
            def _():
                wait_write(b)
            _transpose_block(stage[b], tblk[b], _BLK, d)
            write(j, b)

            @pl.when(j + 2 < nb)
            def _():
                read(j + 2, b)

        read(0, 0)
        read(1, 1)

        @pl.loop(0, nb, step=2)
        def _per_blk(j):
            step(j, 0)
            step(j + 1, 1)

        wait_write(0)
        wait_write(1)

        if tail:
            @pl.when(wid == _NW - 1)
            def _tail():
                pltpu.sync_copy(tail_hbm.at[pl.ds(0, tail)],
                                tblk0.at[pl.ds(0, tail)])
                pltpu.sync_copy(tblk0.at[pl.ds(0, tail)],
                                tab_hbm.at[pl.ds(nfull * _BLK, tail)])

    return body(wt, tail128)


def _embed_gather(xt, tab, bsz, h, d):
    bw = bsz // _NW

    @functools.partial(
        pl.kernel,
        out_type=jax.ShapeDtypeStruct((h, d, bsz), jnp.float32),
        mesh=_mesh(),
        scratch_types=[
            pltpu.VMEM((h, bw), jnp.int32),
            pltpu.VMEM((bw, _BLK), jnp.float32),
            pltpu.VMEM((bw, _BLK), jnp.float32),
            pltpu.VMEM((d, bw), jnp.float32),
            pltpu.VMEM((d, bw), jnp.float32),
            pltpu.SemaphoreType.DMA,
            pltpu.SemaphoreType.DMA,
            pltpu.SemaphoreType.DMA,
            pltpu.SemaphoreType.DMA,
        ],
        compiler_params=pltpu.CompilerParams(needs_layout_passes=False),
    )
    def body(xt_hbm, tab_hbm, out_hbm, idxv, pair0, pair1,
             outb0, outb1, gsem0, gsem1, osem0, osem1):
        pair = (pair0, pair1)
        outbuf = (outb0, outb1)
        gsem = (gsem0, gsem1)
        osem = (osem0, osem1)
        wid = _wid()
        b0 = wid * bw

        pltpu.sync_copy(xt_hbm.at[:, pl.ds(b0, bw)], idxv)

        lanes = lax.iota(jnp.int32, _L)
        rowvecs = [lanes + k * _L for k in range(bw // _L)]

        def fire(l, b):
            pltpu.async_copy(tab_hbm.at[idxv.at[l]], pair[b], gsem[b])

        def drain(b):
            pltpu.make_async_copy(
                tab_hbm.at[pl.ds(0, bw)], pair[b], gsem[b]).wait()

        def emit(l, b):
            drain(b)

            @pl.when(l > 1)
            def _():
                pltpu.make_async_copy(
                    outbuf[b], out_hbm.at[0, :, pl.ds(0, bw)],
                    osem[b]).wait()
            _transpose_block(pair[b], outbuf[b], d, bw)
            pltpu.async_copy(
                outbuf[b], out_hbm.at[l, :, pl.ds(b0, bw)], osem[b])

        fire(0, 0)

        @pl.loop(0, h, step=2)
        def _pair_l(l):
            fire(l + 1, 1)
            emit(l, 0)

            @pl.when(l + 2 < h)
            def _():
                fire(l + 2, 0)
            emit(l + 1, 1)

        for b in (0, 1):
            pltpu.make_async_copy(
                outbuf[b], out_hbm.at[0, :, pl.ds(0, bw)], osem[b]).wait()

    return body(xt, tab)


def kernel(x, embed_weights):
    bsz, h = x.shape
    v, d = embed_weights.shape
    xt = x.T.astype(jnp.int32)
    wt = embed_weights.T
    nfull = v // _BLK
    tail128 = jnp.pad(embed_weights[nfull * _BLK:],
                      ((0, _BLK - (v - nfull * _BLK)), (0, _BLK - d)))
    tab = _relayout_table(wt, tail128, v, d)
    o = _embed_gather(xt, tab, bsz, h, d)
    return jnp.transpose(o, (2, 0, 1))
